# baseline (speedup 1.0000x reference)
import jax
import jax.numpy as jnp
from jax import lax
from jax.experimental import pallas as pl
from jax.experimental.pallas import tpu as pltpu
from jax.experimental.pallas import tpu_sc as plsc

_B, _H, _D = 4096, 200, 32
_NC, _NS = 2, 16
_NW = _NC * _NS
_TH = _H // 8
_TB = _B // 128
_DG = _D // 8
_CHUNK = 128
_NCH = _TH * 8
_NBUF = 4
_LA = 2


def _emb_body(idx_hbm, table_hbm, out_hbm, idx_v, buf, tbuf, gsem, wsem):
    w = lax.axis_index("s") * _NC + lax.axis_index("c")
    pltpu.sync_copy(idx_hbm.at[:, w], idx_v)

    def gather_start(j, s):
        pltpu.make_async_copy(
            table_hbm.at[idx_v.at[j // 8, j % 8]], buf.at[s],
            gsem.at[s]).start()

    def gather_start_dyn(j, s):
        pltpu.make_async_copy(
            table_hbm.at[idx_v.at[lax.div(j, 8), lax.rem(j, 8)]], buf.at[s],
            gsem.at[s]).start()

    def gather_wait(s):
        pltpu.make_async_copy(
            table_hbm.at[idx_v.at[0, 0]], buf.at[s], gsem.at[s]).wait()

    def transpose(s):
        rows = lax.iota(jnp.int32, 16)
        for l in range(8):
            ridx = rows + (l * 16)
            for d in range(_D):
                v = plsc.load_gather(buf.at[s], [ridx, jnp.full((16,), d, jnp.int32)])
                tbuf[s, d, pl.ds(l * 16, 16)] = v

    def write_start(h, s):
        for dg in range(_DG):
            pltpu.make_async_copy(
                tbuf.at[s, pl.ds(dg * 8, 8)], out_hbm.at[h, dg, w],
                wsem.at[s]).start()

    def write_wait(s):
        for dg in range(_DG):
            pltpu.make_async_copy(
                tbuf.at[s, pl.ds(dg * 8, 8)], out_hbm.at[0, dg, w],
                wsem.at[s]).wait()

    def step(j, b, first_round, last_round):
        start = gather_start if isinstance(j, int) else gather_start_dyn
        sf = (b + _LA) % _NBUF
        if not last_round:
            if not (first_round and b < _NBUF - _LA):
                write_wait(sf)
            start(j + _LA, sf)
        elif b < _NBUF - _LA:
            write_wait(sf)
            start(j + _LA, sf)
        gather_wait(b)
        transpose(b)
        write_start(j, b)

    for j in range(_LA):
        gather_start(j, j)

    for b in range(_NBUF):
        step(b, b, True, False)

    @pl.loop(_NBUF, _NCH - _NBUF, step=_NBUF)
    def _round(j0):
        for b in range(_NBUF):
            step(j0 + b, b, False, False)

    for b in range(_NBUF):
        step(_NCH - _NBUF + b, b, False, True)

    for s in range(_NBUF):
        write_wait(s)


@jax.jit
def kernel(batch_seq, table):
    idxn = (batch_seq.astype(jnp.int32).T
            .reshape(_TH, 8, _TB, 128).transpose(0, 2, 1, 3))
    k = pl.kernel(
        _emb_body,
        out_type=jax.ShapeDtypeStruct((_H, _DG, _TB, 8, 128), jnp.float32),
        mesh=plsc.VectorSubcoreMesh(core_axis_name="c", subcore_axis_name="s"),
        scratch_types=[
            pltpu.VMEM((_TH, 8, _CHUNK), jnp.int32),
            pltpu.VMEM((_NBUF, _CHUNK, _D), jnp.float32),
            pltpu.VMEM((_NBUF, _D, _CHUNK), jnp.float32),
            pltpu.SemaphoreType.DMA((_NBUF,)),
            pltpu.SemaphoreType.DMA((_NBUF,)),
        ],
        compiler_params=pltpu.CompilerParams(
            use_tc_tiling_on_sc=False, needs_layout_passes=False),
    )
    out5 = k(idxn, table)
    return out5.transpose(2, 4, 0, 1, 3).reshape(_B, _H, _D)


# speedup vs baseline: 1.4217x; 1.4217x over previous
import jax
import jax.numpy as jnp
from jax import lax
from jax.experimental import pallas as pl
from jax.experimental.pallas import tpu as pltpu
from jax.experimental.pallas import tpu_sc as plsc

_B, _H, _D = 4096, 200, 32
_NC, _NS = 2, 16
_NW = _NC * _NS
_TH = _H // 8
_TB = _B // 128
_DG = _D // 8
_CHUNK = 128
_NCH = _TH * 8
_NBUF = 4
_LA = 2


def _emb_body(idx_hbm, table_hbm, out_hbm, idx_v, buf, tbuf, gsem, wsem):
    w = lax.axis_index("s") * _NC + lax.axis_index("c")
    pltpu.sync_copy(idx_hbm.at[:, w], idx_v)

    def gather_start(j, s):
        pltpu.make_async_copy(
            table_hbm.at[idx_v.at[j // 8, j % 8]], buf.at[s],
            gsem.at[s]).start()

    def gather_start_dyn(j, s):
        pltpu.make_async_copy(
            table_hbm.at[idx_v.at[lax.div(j, 8), lax.rem(j, 8)]], buf.at[s],
            gsem.at[s]).start()

    def gather_wait(s):
        pltpu.make_async_copy(
            table_hbm.at[idx_v.at[0, 0]], buf.at[s], gsem.at[s]).wait()

    def transpose(s):
        rows = lax.iota(jnp.int32, 16)
        for l in range(8):
            ridx = rows + (l * 16)
            for dblk in range(0, _D, 16):
                vs = [plsc.load_gather(
                          buf.at[s], [ridx, jnp.full((16,), d, jnp.int32)])
                      for d in range(dblk, dblk + 16)]
                for i, v in enumerate(vs):
                    d = dblk + i
                    tbuf[s, d // 8, d % 8, pl.ds(l * 16, 16)] = v

    def write_start(h, s):
        pltpu.make_async_copy(
            tbuf.at[s], out_hbm.at[h, :, w], wsem.at[s]).start()

    def write_wait(s):
        pltpu.make_async_copy(
            tbuf.at[s], out_hbm.at[0, :, w], wsem.at[s]).wait()

    def step(j, b, first_round, last_round):
        start = gather_start if isinstance(j, int) else gather_start_dyn
        sf = (b + _LA) % _NBUF
        if not last_round:
            if not (first_round and b < _NBUF - _LA):
                write_wait(sf)
            start(j + _LA, sf)
        elif b < _NBUF - _LA:
            write_wait(sf)
            start(j + _LA, sf)
        gather_wait(b)
        transpose(b)
        write_start(j, b)

    for j in range(_LA):
        gather_start(j, j)

    for b in range(_NBUF):
        step(b, b, True, False)

    @pl.loop(_NBUF, _NCH - _NBUF, step=_NBUF)
    def _round(j0):
        for b in range(_NBUF):
            step(j0 + b, b, False, False)

    for b in range(_NBUF):
        step(_NCH - _NBUF + b, b, False, True)

    for s in range(_NBUF):
        write_wait(s)


@jax.jit
def kernel(batch_seq, table):
    idxn = (batch_seq.astype(jnp.int32).T
            .reshape(_TH, 8, _TB, 128).transpose(0, 2, 1, 3))
    k = pl.kernel(
        _emb_body,
        out_type=jax.ShapeDtypeStruct((_H, _DG, _TB, 8, 128), jnp.float32),
        mesh=plsc.VectorSubcoreMesh(core_axis_name="c", subcore_axis_name="s"),
        scratch_types=[
            pltpu.VMEM((_TH, 8, _CHUNK), jnp.int32),
            pltpu.VMEM((_NBUF, _CHUNK, _D), jnp.float32),
            pltpu.VMEM((_NBUF, _DG, 8, _CHUNK), jnp.float32),
            pltpu.SemaphoreType.DMA((_NBUF,)),
            pltpu.SemaphoreType.DMA((_NBUF,)),
        ],
        compiler_params=pltpu.CompilerParams(
            use_tc_tiling_on_sc=False, needs_layout_passes=False),
    )
    out5 = k(idxn, table)
    return out5.transpose(2, 4, 0, 1, 3).reshape(_B, _H, _D)


# speedup vs baseline: 1.4270x; 1.0037x over previous
import jax
import jax.numpy as jnp
from jax import lax
from jax.experimental import pallas as pl
from jax.experimental.pallas import tpu as pltpu
from jax.experimental.pallas import tpu_sc as plsc

_B, _H, _D = 4096, 200, 32
_NC, _NS = 2, 16
_NW = _NC * _NS
_TH = _H // 8
_TB = _B // 128
_DG = _D // 8
_CHUNK = 128
_NCH = _TH * 8
_NBUF = 8
_LA = 5


def _emb_body(idx_hbm, table_hbm, out_hbm, idx_v, buf, tbuf, gsem, wsem):
    w = lax.axis_index("s") * _NC + lax.axis_index("c")
    pltpu.sync_copy(idx_hbm.at[:, w], idx_v)

    def gather_start(j, s):
        pltpu.make_async_copy(
            table_hbm.at[idx_v.at[j // 8, j % 8]], buf.at[s],
            gsem.at[s]).start()

    def gather_start_dyn(j, s):
        pltpu.make_async_copy(
            table_hbm.at[idx_v.at[lax.div(j, 8), lax.rem(j, 8)]], buf.at[s],
            gsem.at[s]).start()

    def gather_wait(s):
        pltpu.make_async_copy(
            table_hbm.at[idx_v.at[0, 0]], buf.at[s], gsem.at[s]).wait()

    def transpose(s):
        rows = lax.iota(jnp.int32, 16)
        for l in range(8):
            ridx = rows + (l * 16)
            for dblk in range(0, _D, 16):
                vs = [plsc.load_gather(
                          buf.at[s], [ridx, jnp.full((16,), d, jnp.int32)])
                      for d in range(dblk, dblk + 16)]
                for i, v in enumerate(vs):
                    d = dblk + i
                    tbuf[s, d // 8, d % 8, pl.ds(l * 16, 16)] = v

    def write_start(h, s):
        pltpu.make_async_copy(
            tbuf.at[s], out_hbm.at[h, :, w], wsem.at[s]).start()

    def write_wait(s):
        pltpu.make_async_copy(
            tbuf.at[s], out_hbm.at[0, :, w], wsem.at[s]).wait()

    for j in range(_LA):
        gather_start(j, j % _NBUF)

    @pl.loop(0, _NCH, step=_NBUF)
    def _round(j0):
        for b in range(_NBUF):
            j = j0 + b
            sf = (b + _LA) % _NBUF

            @pl.when(j + _LA < _NCH)
            def _():
                @pl.when(j + _LA >= _NBUF)
                def _():
                    write_wait(sf)
                gather_start_dyn(j + _LA, sf)

            gather_wait(b)
            transpose(b)
            write_start(j, b)

    for s in range(_NBUF):
        write_wait(s)


@jax.jit
def kernel(batch_seq, table):
    idxn = (batch_seq.astype(jnp.int32).T
            .reshape(_TH, 8, _TB, 128).transpose(0, 2, 1, 3))
    k = pl.kernel(
        _emb_body,
        out_type=jax.ShapeDtypeStruct((_H, _DG, _TB, 8, 128), jnp.float32),
        mesh=plsc.VectorSubcoreMesh(core_axis_name="c", subcore_axis_name="s"),
        scratch_types=[
            pltpu.VMEM((_TH, 8, _CHUNK), jnp.int32),
            pltpu.VMEM((_NBUF, _CHUNK, _D), jnp.float32),
            pltpu.VMEM((_NBUF, _DG, 8, _CHUNK), jnp.float32),
            pltpu.SemaphoreType.DMA((_NBUF,)),
            pltpu.SemaphoreType.DMA((_NBUF,)),
        ],
        compiler_params=pltpu.CompilerParams(
            use_tc_tiling_on_sc=False, needs_layout_passes=False),
    )
    out5 = k(idxn, table)
    return out5.transpose(2, 4, 0, 1, 3).reshape(_B, _H, _D)
